# baseline (speedup 1.0000x reference)
import functools

import jax
import jax.numpy as jnp
from jax import lax
from jax.experimental import pallas as pl
from jax.experimental.pallas import tpu as pltpu
from jax.experimental.pallas import tpu_sc as plsc

B = 128
SEG = 32768
K = 10
L = 16
NC = 2
NS = 16
NW = NC * NS
SEG_PER_W = B // NW
ROWS = SEG // L
CHUNK_ROWS = 16
NCHUNK = ROWS // CHUNK_ROWS
CHUNK = CHUNK_ROWS * L
GROUP = 16
NGROUP = NCHUNK // GROUP
BIG = 2**30

_MESH = plsc.VectorSubcoreMesh(core_axis_name="c", subcore_axis_name="s")


def _gat(x, idx):
    return lax.gather(
        x, idx.reshape(L, 1),
        lax.GatherDimensionNumbers(
            offset_dims=(), collapsed_slice_dims=(0,), start_index_map=(0,)),
        (1,), unique_indices=False, indices_are_sorted=False,
        mode=lax.GatherScatterMode.PROMISE_IN_BOUNDS)


def _allmax(x, iot):
    for sh in (8, 4, 2, 1):
        x = jnp.maximum(x, _gat(x, iot ^ sh))
    return x


def _allmin(x, iot):
    for sh in (8, 4, 2, 1):
        x = jnp.minimum(x, _gat(x, iot ^ sh))
    return x


_TC_GRID = 16
_TC_BLK = (B * SEG) // _TC_GRID


def _tc_stats_body(x_ref, mx_ref, mn_ref):
    i = pl.program_id(0)

    @pl.when(i == 0)
    def _():
        mx_ref[...] = jnp.full((8, 128), -jnp.inf, jnp.float32)
        mn_ref[...] = jnp.full((8, 128), jnp.inf, jnp.float32)

    x = x_ref[...].reshape(_TC_BLK // (8 * 128), 8, 128)
    mx_ref[...] = jnp.maximum(mx_ref[...], jnp.max(x, axis=0))
    mn_ref[...] = jnp.minimum(mn_ref[...], jnp.min(x, axis=0))


_tc_stats = pl.pallas_call(
    _tc_stats_body,
    grid=(_TC_GRID,),
    in_specs=[pl.BlockSpec((_TC_BLK // 128, 128), lambda i: (i, 0))],
    out_specs=[pl.BlockSpec((8, 128), lambda i: (0, 0)),
               pl.BlockSpec((8, 128), lambda i: (0, 0))],
    out_shape=[jax.ShapeDtypeStruct((8, 128), jnp.float32),
               jax.ShapeDtypeStruct((8, 128), jnp.float32)],
    compiler_params=pltpu.CompilerParams(
        dimension_semantics=("arbitrary",)),
)


@functools.partial(
    pl.kernel,
    out_type=(
        jax.ShapeDtypeStruct((B * L,), jnp.float32),
        jax.ShapeDtypeStruct((B * L,), jnp.int32),
    ),
    mesh=_MESH,
    scratch_types=[
        pltpu.VMEM((SEG,), jnp.float32),
        pltpu.VMEM((SEG,), jnp.float32),
        pltpu.VMEM((NCHUNK * L,), jnp.float32),
        pltpu.VMEM((NGROUP * L,), jnp.float32),
        pltpu.VMEM((1024,), jnp.float32),
        pltpu.VMEM((1024,), jnp.float32),
        pltpu.VMEM((SEG_PER_W * L,), jnp.float32),
        pltpu.VMEM((SEG_PER_W * L,), jnp.int32),
        pltpu.SemaphoreType.DMA,
        pltpu.SemaphoreType.DMA,
    ],
)
def _topk_kernel(x_hbm, mx_hbm, mn_hbm, outv_hbm, outi_hbm,
                 buf0, buf1, mref, g2v, stx, stn, outv, outi, sem0, sem1):
    wid = lax.axis_index("s") * NC + lax.axis_index("c")
    seg0 = wid * SEG_PER_W
    bufs = (buf0, buf1)
    sems = (sem0, sem1)
    iot = lax.iota(jnp.int32, L)

    pend = pltpu.async_copy(x_hbm.at[pl.ds(seg0 * SEG, SEG)], buf0, sem0)

    pltpu.sync_copy(mx_hbm, stx)
    pltpu.sync_copy(mn_hbm, stn)
    def cbody(j, carry):
        cx, cn = carry
        cx = jnp.maximum(cx, stx[pl.ds(j * L, L)])
        cn = jnp.minimum(cn, stn[pl.ds(j * L, L)])
        return cx, cn

    mxv, mnv = lax.fori_loop(
        0, 1024 // L, cbody,
        (jnp.full((L,), -jnp.inf, jnp.float32),
         jnp.full((L,), jnp.inf, jnp.float32)), unroll=8)
    mxv = _allmax(mxv, iot)
    mnv = _allmin(mnv, iot)
    neg_off = -((mxv - mnv) * jnp.float32(4.0))

    for s in range(SEG_PER_W):
        buf = bufs[s % 2]
        seg = seg0 + s
        t = neg_off * seg.astype(jnp.float32)
        pend.wait()
        if s + 1 < SEG_PER_W:
            pend = pltpu.async_copy(
                x_hbm.at[pl.ds((seg + 1) * SEG, SEG)],
                bufs[(s + 1) % 2], sems[(s + 1) % 2])

        def abody(c, _, buf=buf, t=t):
            acc = [jnp.full((L,), -jnp.inf, jnp.float32) for _ in range(4)]
            base = c * CHUNK
            for r in range(CHUNK_ROWS):
                acc[r % 4] = jnp.maximum(acc[r % 4], buf[pl.ds(base + r * L, L)])
            m = jnp.maximum(jnp.maximum(acc[0], acc[1]),
                            jnp.maximum(acc[2], acc[3]))
            mref[pl.ds(c * L, L)] = m + t
            return 0

        lax.fori_loop(0, NCHUNK, abody, 0, unroll=4)

        def gbody(gg, _):
            bv = jnp.full((L,), -jnp.inf, jnp.float32)
            for cc in range(GROUP):
                bv = jnp.maximum(bv, mref[pl.ds((gg * GROUP + cc) * L, L)])
            g2v[pl.ds(gg * L, L)] = bv
            return 0

        lax.fori_loop(0, NGROUP, gbody, 0)

        def ebody(kk, _, buf=buf, t=t):
            bv = jnp.full((L,), -jnp.inf, jnp.float32)
            bg = jnp.zeros((L,), jnp.int32)
            for gg in range(NGROUP):
                val = g2v[pl.ds(gg * L, L)]
                upd = val > bv
                bv = jnp.where(upd, val, bv)
                bg = jnp.where(upd, jnp.int32(gg), bg)
            vmax = _allmax(bv, iot)
            gstar = _allmin(jnp.where(bv == vmax, bg, BIG), iot)[0]
            cbase0 = gstar * GROUP

            bm = jnp.full((L,), -jnp.inf, jnp.float32)
            bc = jnp.zeros((L,), jnp.int32)
            for cc in range(GROUP):
                m = mref[pl.ds((cbase0 + cc) * L, L)]
                upd = m > bm
                bm = jnp.where(upd, m, bm)
                bc = jnp.where(upd, jnp.int32(cc), bc)
            cstar = _allmin(jnp.where(bm == vmax, bc + cbase0, BIG), iot)[0]
            cbase = cstar * CHUNK

            rbest = jnp.full((L,), CHUNK_ROWS, jnp.int32)
            cnt = jnp.zeros((L,), jnp.int32)
            m2 = jnp.full((L,), -jnp.inf, jnp.float32)
            for r in range(CHUNK_ROWS - 1, -1, -1):
                key = buf[pl.ds(cbase + r * L, L)] + t
                eq = key == vmax
                rbest = jnp.where(eq, jnp.int32(r), rbest)
                cnt = cnt + jnp.where(eq, 1, 0)
                m2 = jnp.maximum(m2, jnp.where(eq, jnp.float32(-jnp.inf), key))
            gidx = cbase + rbest * L + iot
            gv = _allmin(
                jnp.where(rbest < CHUNK_ROWS, gidx, BIG), iot)
            g = gv[0]

            rowbase = (g // L) * L
            lane = g - rowbase
            vv = buf[pl.ds(rowbase, L)]
            selm = iot == lane
            valv = _gat(vv, jnp.full((L,), lane, jnp.int32))
            buf[pl.ds(rowbase, L)] = jnp.where(
                selm, jnp.float32(-jnp.inf), vv)

            selk = iot == kk
            outv[pl.ds(s * L, L)] = jnp.where(selk, valv, outv[pl.ds(s * L, L)])
            outi[pl.ds(s * L, L)] = jnp.where(selk, gv, outi[pl.ds(s * L, L)])

            old = mref[pl.ds(cstar * L, L)]
            newlane = jnp.where(cnt > 1, vmax, m2)
            mref[pl.ds(cstar * L, L)] = jnp.where(selm, newlane, old)
            bv2 = jnp.full((L,), -jnp.inf, jnp.float32)
            for cc in range(GROUP):
                bv2 = jnp.maximum(bv2, mref[pl.ds((cbase0 + cc) * L, L)])
            g2v[pl.ds(gstar * L, L)] = bv2
            return 0

        lax.fori_loop(0, K, ebody, 0)

    pltpu.sync_copy(outv, outv_hbm.at[pl.ds(wid * SEG_PER_W * L, SEG_PER_W * L)])
    pltpu.sync_copy(outi, outi_hbm.at[pl.ds(wid * SEG_PER_W * L, SEG_PER_W * L)])


def kernel(input, size, k):
    del size, k
    mx, mn = _tc_stats(input.reshape(B * SEG // 128, 128))
    v, i = _topk_kernel(input, mx.reshape(1024), mn.reshape(1024))
    value = v.reshape(B, L)[:, :K]
    index = i.reshape(B, L)[:, :K]
    return value, index


# speedup vs baseline: 1.0212x; 1.0212x over previous
import functools

import jax
import jax.numpy as jnp
from jax import lax
from jax.experimental import pallas as pl
from jax.experimental.pallas import tpu as pltpu
from jax.experimental.pallas import tpu_sc as plsc

B = 128
SEG = 32768
K = 10
L = 16
NC = 2
NS = 16
NW = NC * NS
SEG_PER_W = B // NW
ROWS = SEG // L
CHUNK_ROWS = 16
NCHUNK = ROWS // CHUNK_ROWS
CHUNK = CHUNK_ROWS * L
GROUP = 16
NGROUP = NCHUNK // GROUP
BIG = 2**30

_MESH = plsc.VectorSubcoreMesh(core_axis_name="c", subcore_axis_name="s")


def _gat(x, idx):
    return lax.gather(
        x, idx.reshape(L, 1),
        lax.GatherDimensionNumbers(
            offset_dims=(), collapsed_slice_dims=(0,), start_index_map=(0,)),
        (1,), unique_indices=False, indices_are_sorted=False,
        mode=lax.GatherScatterMode.PROMISE_IN_BOUNDS)


def _allmax(x, iot):
    for sh in (8, 4, 2, 1):
        x = jnp.maximum(x, _gat(x, iot ^ sh))
    return x


def _allmin(x, iot):
    for sh in (8, 4, 2, 1):
        x = jnp.minimum(x, _gat(x, iot ^ sh))
    return x


_TC_GRID = 16
_TC_BLK = (B * SEG) // _TC_GRID


def _tc_stats_body(x_ref, mx_ref, mn_ref):
    i = pl.program_id(0)

    @pl.when(i == 0)
    def _():
        mx_ref[...] = jnp.full((128,), -jnp.inf, jnp.float32)
        mn_ref[...] = jnp.full((128,), jnp.inf, jnp.float32)

    x = x_ref[...].reshape(_TC_BLK // 128, 128)
    mx_ref[...] = jnp.maximum(mx_ref[...], jnp.max(x, axis=0))
    mn_ref[...] = jnp.minimum(mn_ref[...], jnp.min(x, axis=0))


_tc_stats = pl.pallas_call(
    _tc_stats_body,
    grid=(_TC_GRID,),
    in_specs=[pl.BlockSpec((_TC_BLK,), lambda i: (i,))],
    out_specs=[pl.BlockSpec((128,), lambda i: (0,)),
               pl.BlockSpec((128,), lambda i: (0,))],
    out_shape=[jax.ShapeDtypeStruct((128,), jnp.float32),
               jax.ShapeDtypeStruct((128,), jnp.float32)],
    compiler_params=pltpu.CompilerParams(
        dimension_semantics=("arbitrary",)),
)


@functools.partial(
    pl.kernel,
    out_type=(
        jax.ShapeDtypeStruct((B * L,), jnp.float32),
        jax.ShapeDtypeStruct((B * L,), jnp.int32),
    ),
    mesh=_MESH,
    scratch_types=[
        pltpu.VMEM((SEG,), jnp.float32),
        pltpu.VMEM((SEG,), jnp.float32),
        pltpu.VMEM((NCHUNK * L,), jnp.float32),
        pltpu.VMEM((NGROUP * L,), jnp.float32),
        pltpu.VMEM((128,), jnp.float32),
        pltpu.VMEM((128,), jnp.float32),
        pltpu.VMEM((SEG_PER_W * L,), jnp.float32),
        pltpu.VMEM((SEG_PER_W * L,), jnp.int32),
        pltpu.SemaphoreType.DMA,
        pltpu.SemaphoreType.DMA,
    ],
)
def _topk_kernel(x_hbm, mx_hbm, mn_hbm, outv_hbm, outi_hbm,
                 buf0, buf1, mref, g2v, stx, stn, outv, outi, sem0, sem1):
    wid = lax.axis_index("s") * NC + lax.axis_index("c")
    seg0 = wid * SEG_PER_W
    bufs = (buf0, buf1)
    sems = (sem0, sem1)
    iot = lax.iota(jnp.int32, L)

    pend = pltpu.async_copy(x_hbm.at[pl.ds(seg0 * SEG, SEG)], buf0, sem0)

    pltpu.sync_copy(mx_hbm, stx)
    pltpu.sync_copy(mn_hbm, stn)
    def cbody(j, carry):
        cx, cn = carry
        cx = jnp.maximum(cx, stx[pl.ds(j * L, L)])
        cn = jnp.minimum(cn, stn[pl.ds(j * L, L)])
        return cx, cn

    mxv, mnv = lax.fori_loop(
        0, 128 // L, cbody,
        (jnp.full((L,), -jnp.inf, jnp.float32),
         jnp.full((L,), jnp.inf, jnp.float32)), unroll=8)
    mxv = _allmax(mxv, iot)
    mnv = _allmin(mnv, iot)
    neg_off = -((mxv - mnv) * jnp.float32(4.0))

    for s in range(SEG_PER_W):
        buf = bufs[s % 2]
        seg = seg0 + s
        t = neg_off * seg.astype(jnp.float32)
        pend.wait()
        if s + 1 < SEG_PER_W:
            pend = pltpu.async_copy(
                x_hbm.at[pl.ds((seg + 1) * SEG, SEG)],
                bufs[(s + 1) % 2], sems[(s + 1) % 2])

        def abody(c, _, buf=buf, t=t):
            acc = [jnp.full((L,), -jnp.inf, jnp.float32) for _ in range(4)]
            base = c * CHUNK
            for r in range(CHUNK_ROWS):
                acc[r % 4] = jnp.maximum(acc[r % 4], buf[pl.ds(base + r * L, L)])
            m = jnp.maximum(jnp.maximum(acc[0], acc[1]),
                            jnp.maximum(acc[2], acc[3]))
            mref[pl.ds(c * L, L)] = m + t
            return 0

        lax.fori_loop(0, NCHUNK, abody, 0, unroll=4)

        def gbody(gg, _):
            bv = jnp.full((L,), -jnp.inf, jnp.float32)
            for cc in range(GROUP):
                bv = jnp.maximum(bv, mref[pl.ds((gg * GROUP + cc) * L, L)])
            g2v[pl.ds(gg * L, L)] = bv
            return 0

        lax.fori_loop(0, NGROUP, gbody, 0)

        def ebody(kk, _, buf=buf, t=t):
            bv = jnp.full((L,), -jnp.inf, jnp.float32)
            bg = jnp.zeros((L,), jnp.int32)
            for gg in range(NGROUP):
                val = g2v[pl.ds(gg * L, L)]
                upd = val > bv
                bv = jnp.where(upd, val, bv)
                bg = jnp.where(upd, jnp.int32(gg), bg)
            vmax = _allmax(bv, iot)
            gstar = _allmin(jnp.where(bv == vmax, bg, BIG), iot)[0]
            cbase0 = gstar * GROUP

            bm = jnp.full((L,), -jnp.inf, jnp.float32)
            bc = jnp.zeros((L,), jnp.int32)
            for cc in range(GROUP):
                m = mref[pl.ds((cbase0 + cc) * L, L)]
                upd = m > bm
                bm = jnp.where(upd, m, bm)
                bc = jnp.where(upd, jnp.int32(cc), bc)
            cstar = _allmin(jnp.where(bm == vmax, bc + cbase0, BIG), iot)[0]
            cbase = cstar * CHUNK

            rbest = jnp.full((L,), CHUNK_ROWS, jnp.int32)
            cnt = jnp.zeros((L,), jnp.int32)
            m2 = jnp.full((L,), -jnp.inf, jnp.float32)
            for r in range(CHUNK_ROWS - 1, -1, -1):
                key = buf[pl.ds(cbase + r * L, L)] + t
                eq = key == vmax
                rbest = jnp.where(eq, jnp.int32(r), rbest)
                cnt = cnt + jnp.where(eq, 1, 0)
                m2 = jnp.maximum(m2, jnp.where(eq, jnp.float32(-jnp.inf), key))
            gidx = cbase + rbest * L + iot
            gv = _allmin(
                jnp.where(rbest < CHUNK_ROWS, gidx, BIG), iot)
            g = gv[0]

            rowbase = (g // L) * L
            lane = g - rowbase
            vv = buf[pl.ds(rowbase, L)]
            selm = iot == lane
            valv = _gat(vv, jnp.full((L,), lane, jnp.int32))
            buf[pl.ds(rowbase, L)] = jnp.where(
                selm, jnp.float32(-jnp.inf), vv)

            selk = iot == kk
            outv[pl.ds(s * L, L)] = jnp.where(selk, valv, outv[pl.ds(s * L, L)])
            outi[pl.ds(s * L, L)] = jnp.where(selk, gv, outi[pl.ds(s * L, L)])

            old = mref[pl.ds(cstar * L, L)]
            newlane = jnp.where(cnt > 1, vmax, m2)
            mref[pl.ds(cstar * L, L)] = jnp.where(selm, newlane, old)
            bv2 = jnp.full((L,), -jnp.inf, jnp.float32)
            for cc in range(GROUP):
                bv2 = jnp.maximum(bv2, mref[pl.ds((cbase0 + cc) * L, L)])
            g2v[pl.ds(gstar * L, L)] = bv2
            return 0

        lax.fori_loop(0, K, ebody, 0)

    pltpu.sync_copy(outv, outv_hbm.at[pl.ds(wid * SEG_PER_W * L, SEG_PER_W * L)])
    pltpu.sync_copy(outi, outi_hbm.at[pl.ds(wid * SEG_PER_W * L, SEG_PER_W * L)])


def kernel(input, size, k):
    del size, k
    mx, mn = _tc_stats(input)
    v, i = _topk_kernel(input, mx, mn)
    value = v.reshape(B, L)[:, :K]
    index = i.reshape(B, L)[:, :K]
    return value, index


# speedup vs baseline: 1.0974x; 1.0747x over previous
import functools

import jax
import jax.numpy as jnp
from jax import lax
from jax.experimental import pallas as pl
from jax.experimental.pallas import tpu as pltpu
from jax.experimental.pallas import tpu_sc as plsc

B = 128
SEG = 32768
K = 10
L = 16
NC = 2
NS = 16
NW = NC * NS
SEG_PER_W = B // NW
ROWS = SEG // L
CHUNK_ROWS = 16
NCHUNK = ROWS // CHUNK_ROWS
CHUNK = CHUNK_ROWS * L
GROUP = 16
NGROUP = NCHUNK // GROUP
BIG = 2**30

_MESH = plsc.VectorSubcoreMesh(core_axis_name="c", subcore_axis_name="s")


def _gat(x, idx):
    return lax.gather(
        x, idx.reshape(L, 1),
        lax.GatherDimensionNumbers(
            offset_dims=(), collapsed_slice_dims=(0,), start_index_map=(0,)),
        (1,), unique_indices=False, indices_are_sorted=False,
        mode=lax.GatherScatterMode.PROMISE_IN_BOUNDS)


def _allmax(x, iot):
    for sh in (8, 4, 2, 1):
        x = jnp.maximum(x, _gat(x, iot ^ sh))
    return x


def _allmin(x, iot):
    for sh in (8, 4, 2, 1):
        x = jnp.minimum(x, _gat(x, iot ^ sh))
    return x


_TC_GRID = 8
_TC_BLK = (B * SEG) // _TC_GRID


def _tc_stats_body(x_ref, mx_ref, mn_ref):
    i = pl.program_id(0)

    @pl.when(i == 0)
    def _():
        mx_ref[...] = jnp.full((128,), -jnp.inf, jnp.float32)
        mn_ref[...] = jnp.full((128,), jnp.inf, jnp.float32)

    x = x_ref[...].reshape(_TC_BLK // 128, 128)
    mx_ref[...] = jnp.maximum(mx_ref[...], jnp.max(x, axis=0))
    mn_ref[...] = jnp.minimum(mn_ref[...], jnp.min(x, axis=0))


_tc_stats = pl.pallas_call(
    _tc_stats_body,
    grid=(_TC_GRID,),
    in_specs=[pl.BlockSpec((_TC_BLK,), lambda i: (i,))],
    out_specs=[pl.BlockSpec((128,), lambda i: (0,)),
               pl.BlockSpec((128,), lambda i: (0,))],
    out_shape=[jax.ShapeDtypeStruct((128,), jnp.float32),
               jax.ShapeDtypeStruct((128,), jnp.float32)],
    compiler_params=pltpu.CompilerParams(
        dimension_semantics=("arbitrary",)),
)


@functools.partial(
    pl.kernel,
    out_type=(
        jax.ShapeDtypeStruct((B * L,), jnp.float32),
        jax.ShapeDtypeStruct((B * L,), jnp.int32),
    ),
    mesh=_MESH,
    scratch_types=[
        pltpu.VMEM((SEG,), jnp.float32),
        pltpu.VMEM((SEG,), jnp.float32),
        pltpu.VMEM((NCHUNK * L,), jnp.float32),
        pltpu.VMEM((NGROUP * L,), jnp.float32),
        pltpu.VMEM((128,), jnp.float32),
        pltpu.VMEM((128,), jnp.float32),
        pltpu.VMEM((SEG_PER_W * L,), jnp.float32),
        pltpu.VMEM((SEG_PER_W * L,), jnp.int32),
        pltpu.SemaphoreType.DMA,
        pltpu.SemaphoreType.DMA,
    ],
)
def _topk_kernel(x_hbm, mx_hbm, mn_hbm, outv_hbm, outi_hbm,
                 buf0, buf1, mref, g2v, stx, stn, outv, outi, sem0, sem1):
    wid = lax.axis_index("s") * NC + lax.axis_index("c")
    seg0 = wid * SEG_PER_W
    bufs = (buf0, buf1)
    sems = (sem0, sem1)
    iot = lax.iota(jnp.int32, L)

    pend = pltpu.async_copy(x_hbm.at[pl.ds(seg0 * SEG, SEG)], buf0, sem0)

    pltpu.sync_copy(mx_hbm, stx)
    pltpu.sync_copy(mn_hbm, stn)
    def cbody(j, carry):
        cx, cn = carry
        cx = jnp.maximum(cx, stx[pl.ds(j * L, L)])
        cn = jnp.minimum(cn, stn[pl.ds(j * L, L)])
        return cx, cn

    mxv, mnv = lax.fori_loop(
        0, 128 // L, cbody,
        (jnp.full((L,), -jnp.inf, jnp.float32),
         jnp.full((L,), jnp.inf, jnp.float32)), unroll=8)
    mxv = _allmax(mxv, iot)
    mnv = _allmin(mnv, iot)
    neg_off = -((mxv - mnv) * jnp.float32(4.0))

    for s in range(SEG_PER_W):
        buf = bufs[s % 2]
        seg = seg0 + s
        t = neg_off * seg.astype(jnp.float32)
        pend.wait()
        if s + 1 < SEG_PER_W:
            pend = pltpu.async_copy(
                x_hbm.at[pl.ds((seg + 1) * SEG, SEG)],
                bufs[(s + 1) % 2], sems[(s + 1) % 2])

        def abody(c, _, buf=buf, t=t):
            acc = [jnp.full((L,), -jnp.inf, jnp.float32) for _ in range(4)]
            base = c * CHUNK
            for r in range(CHUNK_ROWS):
                acc[r % 4] = jnp.maximum(acc[r % 4], buf[pl.ds(base + r * L, L)])
            m = jnp.maximum(jnp.maximum(acc[0], acc[1]),
                            jnp.maximum(acc[2], acc[3]))
            mref[pl.ds(c * L, L)] = m + t
            return 0

        lax.fori_loop(0, NCHUNK, abody, 0, unroll=4)

        def gbody(gg, _):
            bv = jnp.full((L,), -jnp.inf, jnp.float32)
            for cc in range(GROUP):
                bv = jnp.maximum(bv, mref[pl.ds((gg * GROUP + cc) * L, L)])
            g2v[pl.ds(gg * L, L)] = bv
            return 0

        lax.fori_loop(0, NGROUP, gbody, 0)

        def ebody(kk, _, buf=buf, t=t):
            bv = jnp.full((L,), -jnp.inf, jnp.float32)
            bg = jnp.zeros((L,), jnp.int32)
            for gg in range(NGROUP):
                val = g2v[pl.ds(gg * L, L)]
                upd = val > bv
                bv = jnp.where(upd, val, bv)
                bg = jnp.where(upd, jnp.int32(gg), bg)
            vmax = _allmax(bv, iot)
            gstar = _allmin(jnp.where(bv == vmax, bg, BIG), iot)[0]
            cbase0 = gstar * GROUP

            bm = jnp.full((L,), -jnp.inf, jnp.float32)
            bc = jnp.zeros((L,), jnp.int32)
            for cc in range(GROUP):
                m = mref[pl.ds((cbase0 + cc) * L, L)]
                upd = m > bm
                bm = jnp.where(upd, m, bm)
                bc = jnp.where(upd, jnp.int32(cc), bc)
            cstar = _allmin(jnp.where(bm == vmax, bc + cbase0, BIG), iot)[0]
            cbase = cstar * CHUNK

            rbest = jnp.full((L,), CHUNK_ROWS, jnp.int32)
            cnt = jnp.zeros((L,), jnp.int32)
            m2 = jnp.full((L,), -jnp.inf, jnp.float32)
            for r in range(CHUNK_ROWS - 1, -1, -1):
                key = buf[pl.ds(cbase + r * L, L)] + t
                eq = key == vmax
                rbest = jnp.where(eq, jnp.int32(r), rbest)
                cnt = cnt + jnp.where(eq, 1, 0)
                m2 = jnp.maximum(m2, jnp.where(eq, jnp.float32(-jnp.inf), key))
            gidx = cbase + rbest * L + iot
            gv = _allmin(
                jnp.where(rbest < CHUNK_ROWS, gidx, BIG), iot)
            g = gv[0]

            rowbase = (g // L) * L
            lane = g - rowbase
            vv = buf[pl.ds(rowbase, L)]
            selm = iot == lane
            valv = _gat(vv, jnp.full((L,), lane, jnp.int32))
            buf[pl.ds(rowbase, L)] = jnp.where(
                selm, jnp.float32(-jnp.inf), vv)

            selk = iot == kk
            outv[pl.ds(s * L, L)] = jnp.where(selk, valv, outv[pl.ds(s * L, L)])
            outi[pl.ds(s * L, L)] = jnp.where(selk, gv, outi[pl.ds(s * L, L)])

            old = mref[pl.ds(cstar * L, L)]
            newlane = jnp.where(cnt > 1, vmax, m2)
            mref[pl.ds(cstar * L, L)] = jnp.where(selm, newlane, old)
            bv2 = jnp.full((L,), -jnp.inf, jnp.float32)
            for cc in range(GROUP):
                bv2 = jnp.maximum(bv2, mref[pl.ds((cbase0 + cc) * L, L)])
            g2v[pl.ds(gstar * L, L)] = bv2
            return 0

        lax.fori_loop(0, K, ebody, 0)

    pltpu.sync_copy(outv, outv_hbm.at[pl.ds(wid * SEG_PER_W * L, SEG_PER_W * L)])
    pltpu.sync_copy(outi, outi_hbm.at[pl.ds(wid * SEG_PER_W * L, SEG_PER_W * L)])


def kernel(input, size, k):
    del size, k
    mx, mn = _tc_stats(input)
    v, i = _topk_kernel(input, mx, mn)
    value = v.reshape(B, L)[:, :K]
    index = i.reshape(B, L)[:, :K]
    return value, index


# speedup vs baseline: 1.1339x; 1.0332x over previous
import functools

import jax
import jax.numpy as jnp
from jax import lax
from jax.experimental import pallas as pl
from jax.experimental.pallas import tpu as pltpu
from jax.experimental.pallas import tpu_sc as plsc

B = 128
SEG = 32768
K = 10
L = 16
NC = 2
NS = 16
NW = NC * NS
SEG_PER_W = B // NW
ROWS = SEG // L
CHUNK_ROWS = 16
NCHUNK = ROWS // CHUNK_ROWS
CHUNK = CHUNK_ROWS * L
GROUP = 16
NGROUP = NCHUNK // GROUP
BIG = 2**30

_MESH = plsc.VectorSubcoreMesh(core_axis_name="c", subcore_axis_name="s")


def _gat(x, idx):
    return lax.gather(
        x, idx.reshape(L, 1),
        lax.GatherDimensionNumbers(
            offset_dims=(), collapsed_slice_dims=(0,), start_index_map=(0,)),
        (1,), unique_indices=False, indices_are_sorted=False,
        mode=lax.GatherScatterMode.PROMISE_IN_BOUNDS)


def _allmax(x, iot):
    for sh in (8, 4, 2, 1):
        x = jnp.maximum(x, _gat(x, iot ^ sh))
    return x


def _allmin(x, iot):
    for sh in (8, 4, 2, 1):
        x = jnp.minimum(x, _gat(x, iot ^ sh))
    return x


_TC_GRID = 4
_TC_BLK = (B * SEG) // _TC_GRID


def _tc_stats_body(x_ref, mx_ref, mn_ref):
    i = pl.program_id(0)

    @pl.when(i == 0)
    def _():
        mx_ref[...] = jnp.full((128,), -jnp.inf, jnp.float32)
        mn_ref[...] = jnp.full((128,), jnp.inf, jnp.float32)

    x = x_ref[...].reshape(_TC_BLK // 128, 128)
    mx_ref[...] = jnp.maximum(mx_ref[...], jnp.max(x, axis=0))
    mn_ref[...] = jnp.minimum(mn_ref[...], jnp.min(x, axis=0))


_tc_stats = pl.pallas_call(
    _tc_stats_body,
    grid=(_TC_GRID,),
    in_specs=[pl.BlockSpec((_TC_BLK,), lambda i: (i,))],
    out_specs=[pl.BlockSpec((128,), lambda i: (0,)),
               pl.BlockSpec((128,), lambda i: (0,))],
    out_shape=[jax.ShapeDtypeStruct((128,), jnp.float32),
               jax.ShapeDtypeStruct((128,), jnp.float32)],
    compiler_params=pltpu.CompilerParams(
        dimension_semantics=("arbitrary",)),
)


@functools.partial(
    pl.kernel,
    out_type=(
        jax.ShapeDtypeStruct((B * L,), jnp.float32),
        jax.ShapeDtypeStruct((B * L,), jnp.int32),
    ),
    mesh=_MESH,
    scratch_types=[
        pltpu.VMEM((SEG,), jnp.float32),
        pltpu.VMEM((SEG,), jnp.float32),
        pltpu.VMEM((NCHUNK * L,), jnp.float32),
        pltpu.VMEM((NGROUP * L,), jnp.float32),
        pltpu.VMEM((128,), jnp.float32),
        pltpu.VMEM((128,), jnp.float32),
        pltpu.VMEM((SEG_PER_W * L,), jnp.float32),
        pltpu.VMEM((SEG_PER_W * L,), jnp.int32),
        pltpu.SemaphoreType.DMA,
        pltpu.SemaphoreType.DMA,
    ],
)
def _topk_kernel(x_hbm, mx_hbm, mn_hbm, outv_hbm, outi_hbm,
                 buf0, buf1, mref, g2v, stx, stn, outv, outi, sem0, sem1):
    wid = lax.axis_index("s") * NC + lax.axis_index("c")
    seg0 = wid * SEG_PER_W
    bufs = (buf0, buf1)
    sems = (sem0, sem1)
    iot = lax.iota(jnp.int32, L)

    pend = pltpu.async_copy(x_hbm.at[pl.ds(seg0 * SEG, SEG)], buf0, sem0)

    pltpu.sync_copy(mx_hbm, stx)
    pltpu.sync_copy(mn_hbm, stn)
    def cbody(j, carry):
        cx, cn = carry
        cx = jnp.maximum(cx, stx[pl.ds(j * L, L)])
        cn = jnp.minimum(cn, stn[pl.ds(j * L, L)])
        return cx, cn

    mxv, mnv = lax.fori_loop(
        0, 128 // L, cbody,
        (jnp.full((L,), -jnp.inf, jnp.float32),
         jnp.full((L,), jnp.inf, jnp.float32)), unroll=8)
    mxv = _allmax(mxv, iot)
    mnv = _allmin(mnv, iot)
    neg_off = -((mxv - mnv) * jnp.float32(4.0))

    for s in range(SEG_PER_W):
        buf = bufs[s % 2]
        seg = seg0 + s
        t = neg_off * seg.astype(jnp.float32)
        pend.wait()
        if s + 1 < SEG_PER_W:
            pend = pltpu.async_copy(
                x_hbm.at[pl.ds((seg + 1) * SEG, SEG)],
                bufs[(s + 1) % 2], sems[(s + 1) % 2])

        def abody(c, _, buf=buf, t=t):
            acc = [jnp.full((L,), -jnp.inf, jnp.float32) for _ in range(4)]
            base = c * CHUNK
            for r in range(CHUNK_ROWS):
                acc[r % 4] = jnp.maximum(acc[r % 4], buf[pl.ds(base + r * L, L)])
            m = jnp.maximum(jnp.maximum(acc[0], acc[1]),
                            jnp.maximum(acc[2], acc[3]))
            mref[pl.ds(c * L, L)] = m + t
            return 0

        lax.fori_loop(0, NCHUNK, abody, 0, unroll=4)

        def gbody(gg, _):
            bv = jnp.full((L,), -jnp.inf, jnp.float32)
            for cc in range(GROUP):
                bv = jnp.maximum(bv, mref[pl.ds((gg * GROUP + cc) * L, L)])
            g2v[pl.ds(gg * L, L)] = bv
            return 0

        lax.fori_loop(0, NGROUP, gbody, 0)

        def ebody(kk, _, buf=buf, t=t):
            bv = jnp.full((L,), -jnp.inf, jnp.float32)
            bg = jnp.zeros((L,), jnp.int32)
            for gg in range(NGROUP):
                val = g2v[pl.ds(gg * L, L)]
                upd = val > bv
                bv = jnp.where(upd, val, bv)
                bg = jnp.where(upd, jnp.int32(gg), bg)
            vmax = _allmax(bv, iot)
            gstar = _allmin(jnp.where(bv == vmax, bg, BIG), iot)[0]
            cbase0 = gstar * GROUP

            bm = jnp.full((L,), -jnp.inf, jnp.float32)
            bc = jnp.zeros((L,), jnp.int32)
            for cc in range(GROUP):
                m = mref[pl.ds((cbase0 + cc) * L, L)]
                upd = m > bm
                bm = jnp.where(upd, m, bm)
                bc = jnp.where(upd, jnp.int32(cc), bc)
            cstar = _allmin(jnp.where(bm == vmax, bc + cbase0, BIG), iot)[0]
            cbase = cstar * CHUNK

            rbest = jnp.full((L,), CHUNK_ROWS, jnp.int32)
            cnt = jnp.zeros((L,), jnp.int32)
            m2 = jnp.full((L,), -jnp.inf, jnp.float32)
            for r in range(CHUNK_ROWS - 1, -1, -1):
                key = buf[pl.ds(cbase + r * L, L)] + t
                eq = key == vmax
                rbest = jnp.where(eq, jnp.int32(r), rbest)
                cnt = cnt + jnp.where(eq, 1, 0)
                m2 = jnp.maximum(m2, jnp.where(eq, jnp.float32(-jnp.inf), key))
            gidx = cbase + rbest * L + iot
            gv = _allmin(
                jnp.where(rbest < CHUNK_ROWS, gidx, BIG), iot)
            g = gv[0]

            rowbase = (g // L) * L
            lane = g - rowbase
            vv = buf[pl.ds(rowbase, L)]
            selm = iot == lane
            valv = _gat(vv, jnp.full((L,), lane, jnp.int32))
            buf[pl.ds(rowbase, L)] = jnp.where(
                selm, jnp.float32(-jnp.inf), vv)

            selk = iot == kk
            outv[pl.ds(s * L, L)] = jnp.where(selk, valv, outv[pl.ds(s * L, L)])
            outi[pl.ds(s * L, L)] = jnp.where(selk, gv, outi[pl.ds(s * L, L)])

            old = mref[pl.ds(cstar * L, L)]
            newlane = jnp.where(cnt > 1, vmax, m2)
            mref[pl.ds(cstar * L, L)] = jnp.where(selm, newlane, old)
            bv2 = jnp.full((L,), -jnp.inf, jnp.float32)
            for cc in range(GROUP):
                bv2 = jnp.maximum(bv2, mref[pl.ds((cbase0 + cc) * L, L)])
            g2v[pl.ds(gstar * L, L)] = bv2
            return 0

        lax.fori_loop(0, K, ebody, 0)

    pltpu.sync_copy(outv, outv_hbm.at[pl.ds(wid * SEG_PER_W * L, SEG_PER_W * L)])
    pltpu.sync_copy(outi, outi_hbm.at[pl.ds(wid * SEG_PER_W * L, SEG_PER_W * L)])


def kernel(input, size, k):
    del size, k
    mx, mn = _tc_stats(input)
    v, i = _topk_kernel(input, mx, mn)
    value = v.reshape(B, L)[:, :K]
    index = i.reshape(B, L)[:, :K]
    return value, index
